# Initial kernel scaffold; baseline (speedup 1.0000x reference)
import jax
import jax.numpy as jnp
from jax.experimental import pallas as pl


def kernel(src, tgt):
    raise NotImplementedError("write your pallas kernel here")


# speedup vs baseline: 21.2112x; 21.2112x over previous
import functools

import jax
import jax.numpy as jnp
from jax import lax
from jax.experimental import pallas as pl
from jax.experimental.pallas import tpu as pltpu
from jax.experimental.pallas import tpu_sc as plsc

H = 512
W = 512
NPIX = H * W
NFLOAT = NPIX * 3
NSUB = 16
NCORE = 2
NW = NSUB * NCORE
A_CHUNK = NFLOAT // NSUB
A_ITERS = A_CHUNK // 48
B_CHUNK = NFLOAT // NW
B_ITERS = B_CHUNK // 48

_MESH = plsc.VectorSubcoreMesh(core_axis_name="c", subcore_axis_name="s")


def _iota16():
    return lax.iota(jnp.int32, 16)


def _hist_kernel(src_hbm, tgt_hbm, cdf_hbm, pixbuf, hist, wbuf, accbuf,
                 bsbuf, cdfout, shist, sbins):
    c = lax.axis_index("c")
    s = lax.axis_index("s")
    lane = _iota16()
    lane3 = lane * 3
    zeros16 = jnp.zeros((16,), jnp.int32)
    ones16 = jnp.ones((16,), jnp.int32)

    for i in range(256):
        hist[pl.ds(i * 16, 16)] = zeros16

    def accumulate(img_ref):
        pltpu.sync_copy(img_ref.at[pl.ds(s * A_CHUNK, A_CHUNK)], pixbuf)

        def body(i, carry):
            off = i * 48
            idxr = lane3 + off
            r = plsc.load_gather(pixbuf, [idxr])
            g = plsc.load_gather(pixbuf, [idxr + 1])
            b = plsc.load_gather(pixbuf, [idxr + 2])
            m = jnp.maximum(jnp.maximum(r, g), b)
            v = ((m + 1.0) * 127.0) * (256.0 / 255.0)
            bin_i = jnp.clip(v.astype(jnp.int32), 0, 255)
            flat = bin_i * 16 + lane
            plsc.addupdate_scatter(hist, [flat], ones16)
            return carry

        lax.fori_loop(0, A_ITERS, body, 0)

    @pl.when(c == 0)
    def _():
        accumulate(src_hbm)

    @pl.when(c == 1)
    def _():
        accumulate(tgt_hbm)

    pltpu.sync_copy(hist, shist.at[s])
    plsc.subcore_barrier()

    for j in range(16):
        accbuf[pl.ds(j * 16, 16)] = zeros16
    for w in range(16):
        pltpu.sync_copy(shist.at[w, pl.ds(s * 256, 256)], wbuf)
        for j in range(16):
            plsc.addupdate(accbuf.at[pl.ds(j * 16, 16)],
                           wbuf[pl.ds(j * 16, 16)])
    binsum = zeros16
    for i in range(16):
        binsum = binsum + plsc.load_gather(accbuf, [lane * 16 + i])
    bsbuf[...] = binsum
    pltpu.sync_copy(bsbuf, sbins.at[pl.ds(s * 16, 16)])
    plsc.subcore_barrier()

    @pl.when(s == 0)
    def _():
        pltpu.sync_copy(sbins, wbuf)
        chunks = []
        carry = jnp.int32(0)
        for j in range(16):
            cs = plsc.cumsum(wbuf[pl.ds(j * 16, 16)]) + carry
            chunks.append(cs)
            carry = cs[15]
        c0 = chunks[0][0]
        for j in range(16):
            val = (chunks[j] - c0).astype(jnp.float32)
            cdfout[pl.ds(j * 16, 16)] = val / jnp.float32(NPIX - 1)
        pltpu.sync_copy(cdfout, cdf_hbm.at[c])


def _map_kernel(src_hbm, cdf_hbm, out_hbm, pixbuf, outbuf, csbuf, ctbuf,
                pxbuf, pxlocal, spx):
    c = lax.axis_index("c")
    s = lax.axis_index("s")
    wid = c * 16 + s
    lane = _iota16()
    lane3 = lane * 3

    pltpu.sync_copy(cdf_hbm.at[0], csbuf)
    pltpu.sync_copy(cdf_hbm.at[1], ctbuf)

    x = csbuf[pl.ds(s * 16, 16)]

    def count_body(j, cnt):
        tv = ctbuf[pl.ds(j * 16, 16)]
        for l in range(16):
            cnt = cnt + jnp.where(tv[l] <= x, 1, 0)
        return cnt

    cnt = lax.fori_loop(0, 16, count_body, jnp.zeros((16,), jnp.int32))
    i1 = jnp.clip(cnt, 1, 255)
    i0 = i1 - 1
    t0 = plsc.load_gather(ctbuf, [i0])
    t1 = plsc.load_gather(ctbuf, [i1])
    d = t1 - t0
    dsafe = jnp.where(d == 0.0, 1.0, d)
    pxv = (i0.astype(jnp.float32) + (x - t0) / dsafe) * jnp.float32(1.0 / 255.0)
    tfirst = ctbuf[pl.ds(0, 16)][0]
    tlast = ctbuf[pl.ds(240, 16)][15]
    pxv = jnp.where(x <= tfirst, 0.0, jnp.where(x >= tlast, 1.0, pxv))
    pxlocal[...] = pxv
    pltpu.sync_copy(pxlocal, spx.at[pl.ds(s * 16, 16)])
    plsc.subcore_barrier()
    pltpu.sync_copy(spx, pxbuf)

    pltpu.sync_copy(src_hbm.at[pl.ds(wid * B_CHUNK, B_CHUNK)], pixbuf)

    def body(i, carry):
        off = i * 48
        idxr = lane3 + off
        r = plsc.load_gather(pixbuf, [idxr])
        g = plsc.load_gather(pixbuf, [idxr + 1])
        b = plsc.load_gather(pixbuf, [idxr + 2])
        m = jnp.maximum(jnp.maximum(r, g), b)
        t = (m + 1.0) * 127.0
        k = jnp.clip(t.astype(jnp.int32), 0, 254)
        frac = t - k.astype(jnp.float32)
        p0 = plsc.load_gather(pxbuf, [k])
        p1 = plsc.load_gather(pxbuf, [k + 1])
        vn = (p0 + (p1 - p0) * frac) * 255.0
        iszero = t == 0.0
        tsafe = jnp.where(iszero, 1.0, t)
        scale = vn / tsafe
        zout = vn * jnp.float32(1.0 / 127.0) - 1.0
        o_r = jnp.where(iszero, zout, (r + 1.0) * scale - 1.0)
        o_g = jnp.where(iszero, zout, (g + 1.0) * scale - 1.0)
        o_b = jnp.where(iszero, zout, (b + 1.0) * scale - 1.0)
        plsc.store_scatter(outbuf, [idxr], o_r)
        plsc.store_scatter(outbuf, [idxr + 1], o_g)
        plsc.store_scatter(outbuf, [idxr + 2], o_b)
        return carry

    lax.fori_loop(0, B_ITERS, body, 0)
    pltpu.sync_copy(outbuf, out_hbm.at[pl.ds(wid * B_CHUNK, B_CHUNK)])


@functools.partial(
    pl.kernel,
    mesh=_MESH,
    compiler_params=pltpu.CompilerParams(needs_layout_passes=False),
    out_type=jax.ShapeDtypeStruct((NCORE, 256), jnp.float32),
    scratch_types=[
        pltpu.VMEM((A_CHUNK,), jnp.float32),
        pltpu.VMEM((4096,), jnp.int32),
        pltpu.VMEM((256,), jnp.int32),
        pltpu.VMEM((256,), jnp.int32),
        pltpu.VMEM((16,), jnp.int32),
        pltpu.VMEM((256,), jnp.float32),
        pltpu.VMEM_SHARED((16, 4096), jnp.int32),
        pltpu.VMEM_SHARED((256,), jnp.int32),
    ],
)
def _hist_cdf(src_hbm, tgt_hbm, cdf_hbm, *scratch):
    _hist_kernel(src_hbm, tgt_hbm, cdf_hbm, *scratch)


@functools.partial(
    pl.kernel,
    mesh=_MESH,
    compiler_params=pltpu.CompilerParams(needs_layout_passes=False),
    out_type=jax.ShapeDtypeStruct((NFLOAT,), jnp.float32),
    scratch_types=[
        pltpu.VMEM((B_CHUNK,), jnp.float32),
        pltpu.VMEM((B_CHUNK,), jnp.float32),
        pltpu.VMEM((256,), jnp.float32),
        pltpu.VMEM((256,), jnp.float32),
        pltpu.VMEM((256,), jnp.float32),
        pltpu.VMEM((16,), jnp.float32),
        pltpu.VMEM_SHARED((256,), jnp.float32),
    ],
)
def _apply_map(src_hbm, cdf_hbm, out_hbm, *scratch):
    _map_kernel(src_hbm, cdf_hbm, out_hbm, *scratch)


def kernel(src, tgt):
    src_flat = src.reshape(NFLOAT)
    tgt_flat = tgt.reshape(NFLOAT)
    cdfs = _hist_cdf(src_flat, tgt_flat)
    out = _apply_map(src_flat, cdfs)
    return out.reshape(H, W, 3)
